# baseline (speedup 1.0000x reference)
import functools

import jax
import jax.numpy as jnp
from jax import lax
from jax.experimental import pallas as pl
from jax.experimental.pallas import tpu as pltpu
from jax.experimental.pallas import tpu_sc as plsc

_H = 512
_W = 512
_L = 6
_HW = _H * _W
_K = _HW // 2
_NR = 20
_CR = 16
_NCHUNK = _H // _CR
_GRP = _CR * _W // 16
_B1 = 16384
_B2 = 65536
_KTAP = 5


def _conv_body(gb_ref, conf_ref, warp_ref, bmat_ref, maps_ref):
    c0 = conf_ref[0, 0, 0]
    c1 = conf_ref[0, 0, 1]
    s = jax.nn.sigmoid(jnp.maximum(c0, c1))
    s = s * warp_ref[0, 0]
    sb = s.astype(jnp.bfloat16)
    zc = jnp.zeros((_H, 2), jnp.bfloat16)
    sp = jnp.concatenate([zc, sb, zc], axis=1)
    zr = jnp.zeros((2, _W + 4), jnp.bfloat16)
    sp = jnp.concatenate([zr, sp, zr], axis=0)
    acc = jnp.full((_H, _W), gb_ref[0], jnp.float32)
    for dy in range(_KTAP):
        acc = acc + jax.lax.dot_general(
            sp[dy:dy + _H, :], bmat_ref[dy],
            (((1,), (0,)), ((), ())),
            preferred_element_type=jnp.float32)
    maps_ref[0] = acc


def _scan_desc(read16, nbins, kr, iot):
    def coarse(d, car):
        acc, dstar, accstar = car
        base = nbins - 64 * (d + 1)
        tot = jnp.int32(0)
        for q in range(4):
            tot = tot + jnp.sum(read16(base + q * 16))
        newacc = acc + tot
        hit = jnp.logical_and(dstar < 0, newacc >= kr)
        return (newacc,
                jnp.where(hit, d, dstar),
                jnp.where(hit, acc, accstar))

    _, dstar, accstar = lax.fori_loop(
        0, nbins // 64, coarse, (jnp.int32(0), jnp.int32(-1), jnp.int32(0)))

    def fine(q, car):
        acc2, jstar, ngt, nge = car
        base = nbins - 64 * (dstar + 1) + (3 - q) * 16
        ch = read16(base)
        rv = lax.rev(ch, (0,))
        cs = plsc.cumsum(rv)
        cond = (acc2 + cs) >= kr
        nf = jnp.sum(jnp.where(cond, 1, 0))
        p = 16 - nf
        csp = jnp.sum(jnp.where(iot == p, cs, 0))
        rvp = jnp.sum(jnp.where(iot == p, rv, 0))
        hit = jnp.logical_and(jstar < 0, nf > 0)
        return (acc2 + jnp.sum(ch),
                jnp.where(hit, base + 15 - p, jstar),
                jnp.where(hit, acc2 + csp - rvp, ngt),
                jnp.where(hit, acc2 + csp, nge))

    _, jstar, ngt, nge = lax.fori_loop(
        0, 4, fine, (accstar, jnp.int32(-1), jnp.int32(0), jnp.int32(0)))
    return jstar, ngt, nge


def _sc_body(maps_hbm, zeros_hbm, out_hbm,
             buf0, buf1, hist1, hist2, outv, sem0, sem1):
    wid = lax.axis_index("s") * 2 + lax.axis_index("c")

    @pl.when(wid < _NR)
    def _():
        pltpu.sync_copy(zeros_hbm.at[pl.ds(0, 2 * _B1)], hist1)
        pltpu.sync_copy(zeros_hbm.at[pl.ds(0, _B2 + 16)], hist2)
        iot = lax.iota(jnp.int32, 16)

        def run_pass(proc_chunk):
            pltpu.async_copy(maps_hbm.at[wid, pl.ds(0, _CR)], buf0, sem0)
            pltpu.async_copy(maps_hbm.at[wid, pl.ds(_CR, _CR)], buf1, sem1)

            def outer(cb, _):
                for par, (buf, sem) in enumerate(((buf0, sem0), (buf1, sem1))):
                    c = 2 * cb + par
                    pltpu.make_async_copy(
                        maps_hbm.at[wid, pl.ds(c * _CR, _CR)], buf, sem).wait()
                    proc_chunk(buf)

                    @pl.when(c + 2 < _NCHUNK)
                    def _():
                        pltpu.async_copy(
                            maps_hbm.at[wid, pl.ds((c + 2) * _CR, _CR)],
                            buf, sem)
                return 0

            lax.fori_loop(0, _NCHUNK // 2, outer, 0)

        def proc1(buf):
            @plsc.parallel_loop(0, _GRP, 2, unroll=4)
            def _loop(t):
                for h in range(2):
                    tt = t + h
                    i = tt >> 5
                    off = (tt & 31) * 16
                    b = plsc.bitcast(buf[i, pl.ds(off, 16)], jnp.int32)
                    k1 = lax.shift_right_logical(b, 16)
                    cnt, last = plsc.scan_count(k1)
                    plsc.addupdate_scatter(
                        hist1, [k1 + h * _B1], cnt, mask=last)

        run_pass(proc1)

        def read1(base):
            return hist1[pl.ds(base, 16)] + hist1[pl.ds(base + _B1, 16)]

        j1, ngt1, _ = _scan_desc(read1, _B1, jnp.int32(_K), iot)

        def proc2(buf):
            @plsc.parallel_loop(0, _GRP, 1, unroll=8)
            def _loop(t):
                i = t >> 5
                off = (t & 31) * 16
                b = plsc.bitcast(buf[i, pl.ds(off, 16)], jnp.int32)
                hi = lax.shift_right_logical(b, 16)
                idx = jnp.where(hi == j1, b & jnp.int32(0xFFFF),
                                jnp.int32(_B2))
                cnt, last = plsc.scan_count(idx)
                plsc.addupdate_scatter(hist2, [idx], cnt, mask=last)

        run_pass(proc2)

        def read2(base):
            return hist2[pl.ds(base, 16)]

        j2, _, nge2 = _scan_desc(read2, _B2, jnp.int32(_K) - ngt1, iot)

        tbits = lax.shift_left(j1, 16) | j2
        cge = (ngt1 + nge2).astype(jnp.float32)
        tvec = plsc.bitcast(jnp.where(iot == 0, tbits, 0), jnp.float32)
        outv[pl.ds(0, 16)] = tvec + jnp.where(iot == 1, cge, jnp.float32(0))
        pltpu.sync_copy(outv, out_hbm.at[wid])


def _mask_body(thr_ref, maps_ref, out_ref):
    i = pl.program_id(0)
    l = i % _L

    @pl.when(l == 0)
    def _():
        out_ref[0] = jnp.ones((_H, _W), jnp.float32)

    @pl.when(l != 0)
    def _():
        t = thr_ref[(i // _L) * (_L - 1) + l - 1]
        out_ref[0] = (maps_ref[0] >= t).astype(jnp.float32)


def _rtne_bf16_f32(x):
    u = lax.bitcast_convert_type(x, jnp.uint32)
    u = (u + jnp.uint32(0x7FFF) + ((u >> 16) & jnp.uint32(1))) & jnp.uint32(0xFFFF0000)
    return lax.bitcast_convert_type(u, jnp.float32)


def kernel(batch_confidence_maps, B, batch_warp_maks_list, gw, gb):
    Bs, L, C, H, W = batch_confidence_maps.shape
    wq = _rtne_bf16_f32(gw.reshape(_KTAP, _KTAP))
    iu = lax.broadcasted_iota(jnp.int32, (_KTAP, _W + 4, _W), 1)
    ix = lax.broadcasted_iota(jnp.int32, (_KTAP, _W + 4, _W), 2)
    d = iu - ix
    idy = lax.broadcasted_iota(jnp.int32, (_KTAP, _W + 4, _W), 0)
    band = jnp.logical_and(d >= 0, d < _KTAP)
    bmat = jnp.where(band, wq[idy, jnp.clip(d, 0, _KTAP - 1)],
                     jnp.float32(0)).astype(jnp.bfloat16)

    maps = pl.pallas_call(
        _conv_body,
        grid=(_NR,),
        in_specs=[
            pl.BlockSpec(memory_space=pltpu.SMEM),
            pl.BlockSpec((1, 1, C, H, W),
                         lambda i: (i // (L - 1), i % (L - 1) + 1, 0, 0, 0)),
            pl.BlockSpec((1, 1, H, W), lambda i: (i // (L - 1), 0, 0, 0)),
            pl.BlockSpec((_KTAP, _W + 4, _W), lambda i: (0, 0, 0)),
        ],
        out_specs=pl.BlockSpec((1, H, W), lambda i: (i, 0, 0)),
        out_shape=jax.ShapeDtypeStruct((_NR, H, W), jnp.float32),
        compiler_params=pltpu.CompilerParams(
            dimension_semantics=("arbitrary",)),
    )(gb, batch_confidence_maps, batch_warp_maks_list, bmat)

    zeros = jnp.zeros((_B2 + 16,), jnp.int32)
    mesh = plsc.VectorSubcoreMesh(core_axis_name="c", subcore_axis_name="s")
    scout = pl.kernel(
        _sc_body,
        out_type=jax.ShapeDtypeStruct((32, 128), jnp.float32),
        mesh=mesh,
        scratch_types=[
            pltpu.VMEM((_CR, _W), jnp.float32),
            pltpu.VMEM((_CR, _W), jnp.float32),
            pltpu.VMEM((2 * _B1,), jnp.int32),
            pltpu.VMEM((_B2 + 16,), jnp.int32),
            pltpu.VMEM((128,), jnp.float32),
            pltpu.SemaphoreType.DMA,
            pltpu.SemaphoreType.DMA,
        ],
        compiler_params=pltpu.CompilerParams(needs_layout_passes=False),
    )(maps, zeros)

    thr = scout[:_NR, 0]
    counts = scout[:_NR, 1]

    masks = pl.pallas_call(
        _mask_body,
        grid=(Bs * L,),
        in_specs=[
            pl.BlockSpec(memory_space=pltpu.SMEM),
            pl.BlockSpec((1, H, W),
                         lambda i: ((i // L) * (L - 1) + jnp.maximum(i % L, 1) - 1,
                                    0, 0)),
        ],
        out_specs=pl.BlockSpec((1, H, W), lambda i: (i, 0, 0)),
        out_shape=jax.ShapeDtypeStruct((Bs * L, H, W), jnp.float32),
        compiler_params=pltpu.CompilerParams(
            dimension_semantics=("arbitrary",)),
    )(thr, maps)

    rate = jnp.sum(counts) / jnp.float32(Bs * (L - 1) * H * W)
    return masks.reshape(Bs * L, 1, H, W), rate


# speedup vs baseline: 110.7990x; 110.7990x over previous
import functools

import jax
import jax.numpy as jnp
from jax import lax
from jax.experimental import pallas as pl
from jax.experimental.pallas import tpu as pltpu
from jax.experimental.pallas import tpu_sc as plsc

_H = 512
_W = 512
_L = 6
_HW = _H * _W
_K = _HW // 2
_NR = 20
_CR = 16
_NCHUNK = _H // _CR
_GRP = _CR * _W // 16
_B1 = 16384
_B2 = 65536
_KTAP = 5


def _conv_body(gb_ref, conf_ref, warp_ref, bmat_ref, maps_ref):
    c0 = conf_ref[0, 0, 0]
    c1 = conf_ref[0, 0, 1]
    s = jax.nn.sigmoid(jnp.maximum(c0, c1))
    s = s * warp_ref[0, 0]
    sb = s.astype(jnp.bfloat16)
    zc = jnp.zeros((_H, 2), jnp.bfloat16)
    sp = jnp.concatenate([zc, sb, zc], axis=1)
    zr = jnp.zeros((2, _W + 4), jnp.bfloat16)
    sp = jnp.concatenate([zr, sp, zr], axis=0)
    acc = jnp.full((_H, _W), gb_ref[0], jnp.float32)
    for dy in range(_KTAP):
        acc = acc + jax.lax.dot_general(
            sp[dy:dy + _H, :], bmat_ref[dy],
            (((1,), (0,)), ((), ())),
            preferred_element_type=jnp.float32)
    maps_ref[0] = acc


def _scan_desc(read16, nbins, kr, iot):
    def coarse(d, car):
        acc, dstar, accstar = car
        base = nbins - 64 * (d + 1)
        tot = jnp.int32(0)
        for q in range(4):
            tot = tot + jnp.sum(read16(base + q * 16))
        newacc = acc + tot
        hit = jnp.logical_and(dstar < 0, newacc >= kr)
        return (newacc,
                jnp.where(hit, d, dstar),
                jnp.where(hit, acc, accstar))

    _, dstar, accstar = lax.fori_loop(
        0, nbins // 64, coarse, (jnp.int32(0), jnp.int32(-1), jnp.int32(0)))

    def fine(q, car):
        acc2, jstar, ngt, nge = car
        base = nbins - 64 * (dstar + 1) + (3 - q) * 16
        ch = read16(base)
        rv = lax.rev(ch, (0,))
        cs = plsc.cumsum(rv)
        cond = (acc2 + cs) >= kr
        nf = jnp.sum(jnp.where(cond, 1, 0))
        p = 16 - nf
        csp = jnp.sum(jnp.where(iot == p, cs, 0))
        rvp = jnp.sum(jnp.where(iot == p, rv, 0))
        hit = jnp.logical_and(jstar < 0, nf > 0)
        return (acc2 + jnp.sum(ch),
                jnp.where(hit, base + 15 - p, jstar),
                jnp.where(hit, acc2 + csp - rvp, ngt),
                jnp.where(hit, acc2 + csp, nge))

    _, jstar, ngt, nge = lax.fori_loop(
        0, 4, fine, (accstar, jnp.int32(-1), jnp.int32(0), jnp.int32(0)))
    return jstar, ngt, nge


def _sc_body(maps_hbm, zeros_hbm, out_hbm,
             buf0, buf1, hist1, hist2, outv, sem0, sem1):
    wid = lax.axis_index("s") * 2 + lax.axis_index("c")

    @pl.when(wid < _NR)
    def _():
        pltpu.sync_copy(zeros_hbm.at[pl.ds(0, 2 * _B1)], hist1)
        pltpu.sync_copy(zeros_hbm.at[pl.ds(0, _B2 + 16)], hist2)
        iot = lax.iota(jnp.int32, 16)

        def run_pass(proc_chunk):
            pltpu.async_copy(maps_hbm.at[wid, pl.ds(0, _CR)], buf0, sem0)
            pltpu.async_copy(maps_hbm.at[wid, pl.ds(_CR, _CR)], buf1, sem1)

            def outer(cb, _):
                for par, (buf, sem) in enumerate(((buf0, sem0), (buf1, sem1))):
                    c = 2 * cb + par
                    pltpu.make_async_copy(
                        maps_hbm.at[wid, pl.ds(c * _CR, _CR)], buf, sem).wait()
                    proc_chunk(buf)

                    @pl.when(c + 2 < _NCHUNK)
                    def _():
                        pltpu.async_copy(
                            maps_hbm.at[wid, pl.ds((c + 2) * _CR, _CR)],
                            buf, sem)
                return 0

            lax.fori_loop(0, _NCHUNK // 2, outer, 0)

        def proc1(buf):
            @plsc.parallel_loop(0, _GRP, 2, unroll=4)
            def _loop(t):
                for h in range(2):
                    tt = t + h
                    i = tt >> 5
                    off = (tt & 31) * 16
                    b = plsc.bitcast(buf[i, pl.ds(off, 16)], jnp.int32)
                    k1 = lax.shift_right_logical(b, 16)
                    cnt, last = plsc.scan_count(k1)
                    plsc.addupdate_scatter(
                        hist1, [k1 + h * _B1], cnt, mask=last)

        run_pass(proc1)

        def read1(base):
            return hist1[pl.ds(base, 16)] + hist1[pl.ds(base + _B1, 16)]

        j1, ngt1, _ = _scan_desc(read1, _B1, jnp.int32(_K), iot)

        def proc2(buf):
            @plsc.parallel_loop(0, _GRP, 1, unroll=8)
            def _loop(t):
                i = t >> 5
                off = (t & 31) * 16
                b = plsc.bitcast(buf[i, pl.ds(off, 16)], jnp.int32)
                hi = lax.shift_right_logical(b, 16)
                idx = jnp.where(hi == j1, b & jnp.int32(0xFFFF),
                                jnp.int32(_B2))
                cnt, last = plsc.scan_count(idx)
                plsc.addupdate_scatter(hist2, [idx], cnt, mask=last)

        run_pass(proc2)

        def read2(base):
            return hist2[pl.ds(base, 16)]

        j2, _, nge2 = _scan_desc(read2, _B2, jnp.int32(_K) - ngt1, iot)

        tbits = lax.shift_left(j1, 16) | j2
        cge = (ngt1 + nge2).astype(jnp.float32)
        tvec = plsc.bitcast(jnp.where(iot == 0, tbits, 0), jnp.float32)
        outv[pl.ds(0, 16)] = tvec + jnp.where(iot == 1, cge, jnp.float32(0))
        pltpu.sync_copy(outv, out_hbm.at[wid])


def _mask_body(thr_ref, maps_ref, out_ref):
    i = pl.program_id(0)
    l = i % _L

    @pl.when(l == 0)
    def _():
        out_ref[0] = jnp.ones((_H, _W), jnp.float32)

    @pl.when(l != 0)
    def _():
        t = thr_ref[(i // _L) * (_L - 1) + l - 1]
        out_ref[0] = (maps_ref[0] >= t).astype(jnp.float32)


def _rtne_bf16_f32(x):
    u = lax.bitcast_convert_type(x, jnp.uint32)
    u = (u + jnp.uint32(0x7FFF) + ((u >> 16) & jnp.uint32(1))) & jnp.uint32(0xFFFF0000)
    return lax.bitcast_convert_type(u, jnp.float32)


def kernel(batch_confidence_maps, B, batch_warp_maks_list, gw, gb):
    Bs, L, C, H, W = batch_confidence_maps.shape
    wq = _rtne_bf16_f32(gw.reshape(_KTAP, _KTAP))
    iu = lax.broadcasted_iota(jnp.int32, (_W + 4, _W), 0)
    ix = lax.broadcasted_iota(jnp.int32, (_W + 4, _W), 1)
    d = iu - ix
    bmat = jnp.zeros((_KTAP, _W + 4, _W), jnp.float32)
    for k in range(_KTAP):
        bmat = bmat + jnp.where(d == k, 1.0, 0.0)[None] * wq[:, k, None, None]
    bmat = bmat.astype(jnp.bfloat16)

    maps = pl.pallas_call(
        _conv_body,
        grid=(_NR,),
        in_specs=[
            pl.BlockSpec(memory_space=pltpu.SMEM),
            pl.BlockSpec((1, 1, C, H, W),
                         lambda i: (i // (L - 1), i % (L - 1) + 1, 0, 0, 0)),
            pl.BlockSpec((1, 1, H, W), lambda i: (i // (L - 1), 0, 0, 0)),
            pl.BlockSpec((_KTAP, _W + 4, _W), lambda i: (0, 0, 0)),
        ],
        out_specs=pl.BlockSpec((1, H, W), lambda i: (i, 0, 0)),
        out_shape=jax.ShapeDtypeStruct((_NR, H, W), jnp.float32),
        compiler_params=pltpu.CompilerParams(
            dimension_semantics=("arbitrary",)),
    )(gb, batch_confidence_maps, batch_warp_maks_list, bmat)

    zeros = jnp.zeros((_B2 + 16,), jnp.int32)
    mesh = plsc.VectorSubcoreMesh(core_axis_name="c", subcore_axis_name="s")
    scout = pl.kernel(
        _sc_body,
        out_type=jax.ShapeDtypeStruct((32, 128), jnp.float32),
        mesh=mesh,
        scratch_types=[
            pltpu.VMEM((_CR, _W), jnp.float32),
            pltpu.VMEM((_CR, _W), jnp.float32),
            pltpu.VMEM((2 * _B1,), jnp.int32),
            pltpu.VMEM((_B2 + 16,), jnp.int32),
            pltpu.VMEM((128,), jnp.float32),
            pltpu.SemaphoreType.DMA,
            pltpu.SemaphoreType.DMA,
        ],
        compiler_params=pltpu.CompilerParams(needs_layout_passes=False),
    )(maps, zeros)

    thr = scout[:_NR, 0]
    counts = scout[:_NR, 1]

    masks = pl.pallas_call(
        _mask_body,
        grid=(Bs * L,),
        in_specs=[
            pl.BlockSpec(memory_space=pltpu.SMEM),
            pl.BlockSpec((1, H, W),
                         lambda i: ((i // L) * (L - 1) + jnp.maximum(i % L, 1) - 1,
                                    0, 0)),
        ],
        out_specs=pl.BlockSpec((1, H, W), lambda i: (i, 0, 0)),
        out_shape=jax.ShapeDtypeStruct((Bs * L, H, W), jnp.float32),
        compiler_params=pltpu.CompilerParams(
            dimension_semantics=("arbitrary",)),
    )(thr, maps)

    rate = jnp.sum(counts) / jnp.float32(Bs * (L - 1) * H * W)
    return masks.reshape(Bs * L, 1, H, W), rate
